# baseline (speedup 1.0000x reference)
import functools

import jax
import jax.numpy as jnp
from jax import lax
from jax.experimental import pallas as pl
from jax.experimental.pallas import tpu as pltpu
from jax.experimental.pallas import tpu_sc as plsc

NUM_CORES = 2
NUM_SUBCORES = 16
NW = NUM_CORES * NUM_SUBCORES
LANES = 16
CH = 32
NB = 3


@functools.cache
def _make_kernel(B, S, D):
    T = B * S
    s_per_w = S // NW
    n_sub = s_per_w // CH
    n_chunks = B * n_sub
    mesh = plsc.VectorSubcoreMesh(core_axis_name="c", subcore_axis_name="s")

    @functools.partial(
        pl.kernel,
        out_type=jax.ShapeDtypeStruct((B, S, D), jnp.float32),
        mesh=mesh,
        scratch_types=[
            pltpu.VMEM((n_chunks, CH), jnp.int32),
            pltpu.VMEM((s_per_w, D), jnp.float32),
        ]
        + [pltpu.VMEM((CH, D), jnp.float32) for _ in range(NB)]
        + [pltpu.SemaphoreType.DMA for _ in range(2 * NB + 1)],
    )
    def k(seq_hbm, tok_hbm, pos_hbm, out_hbm, idx_v, pos_v, *rest):
        bufs = rest[:NB]
        gsems = rest[NB : 2 * NB]
        osems = rest[2 * NB : 3 * NB]
        isem = rest[3 * NB]
        wid = lax.axis_index("s") * NUM_CORES + lax.axis_index("c")
        s0 = wid * s_per_w

        def chunk_bs(i):
            b, sub = divmod(i, n_sub)
            return b, s0 + sub * CH

        def gather(i, kb):
            return pltpu.async_copy(tok_hbm.at[idx_v.at[i]], bufs[kb], gsems[kb])

        def write_out(i, kb):
            b, col = chunk_bs(i)
            return pltpu.async_copy(
                bufs[kb], out_hbm.at[b, pl.ds(col, CH)], osems[kb]
            )

        idx_copies = [
            pltpu.async_copy(
                seq_hbm.at[chunk_bs(i)[0], pl.ds(chunk_bs(i)[1], CH)],
                idx_v.at[i],
                isem,
            )
            for i in range(n_chunks)
        ]
        pos_copy = pltpu.async_copy(pos_hbm.at[pl.ds(s0, s_per_w)], pos_v, isem)
        for c in idx_copies:
            c.wait()
        pend_w = [None] * NB
        pend_g = [None] * NB
        pend_g[0] = gather(0, 0)
        pos_copy.wait()
        for i in range(n_chunks):
            kb = i % NB
            if i + 1 < n_chunks:
                k2 = (i + 1) % NB
                if pend_w[k2] is not None:
                    pend_w[k2].wait()
                pend_g[k2] = gather(i + 1, k2)
            pend_g[kb].wait()

            p0 = (i % n_sub) * CH

            @plsc.parallel_loop(0, CH, step=1, unroll=3)
            def add_row(r, kb=kb, p0=p0):
                for c in range(D // LANES):
                    sl = pl.ds(c * LANES, LANES)
                    plsc.addupdate(bufs[kb].at[r, sl], pos_v[p0 + r, sl])

            pend_w[kb] = write_out(i, kb)
        for kb in range(NB):
            if pend_w[kb] is not None:
                pend_w[kb].wait()

    return k


def kernel(seq, tok_table, pos_table):
    B, S = seq.shape
    V, D = tok_table.shape
    k = _make_kernel(B, S, D)
    return k(seq.astype(jnp.int32), tok_table, pos_table)


# speedup vs baseline: 1.2840x; 1.2840x over previous
import functools

import jax
import jax.numpy as jnp
from jax import lax
from jax.experimental import pallas as pl
from jax.experimental.pallas import tpu as pltpu
from jax.experimental.pallas import tpu_sc as plsc

NUM_CORES = 2
NUM_SUBCORES = 16
NW = NUM_CORES * NUM_SUBCORES
LANES = 16
CH = 32
NB = 3


@functools.cache
def _make_kernel(B, S, D):
    T = B * S
    s_per_w = S // NW
    n_sub = s_per_w // CH
    n_chunks = B * n_sub
    mesh = plsc.VectorSubcoreMesh(core_axis_name="c", subcore_axis_name="s")

    @functools.partial(
        pl.kernel,
        out_type=jax.ShapeDtypeStruct((B, S, D), jnp.float32),
        mesh=mesh,
        scratch_types=[
            pltpu.VMEM((n_chunks, CH), jnp.int32),
            pltpu.VMEM((s_per_w, D), jnp.float32),
        ]
        + [pltpu.VMEM((CH, D), jnp.float32) for _ in range(NB)]
        + [pltpu.SemaphoreType.DMA for _ in range(2 * NB + 1)],
    )
    def k(seq_hbm, tok_hbm, pos_hbm, out_hbm, idx_v, pos_v, *rest):
        bufs = rest[:NB]
        gsems = rest[NB : 2 * NB]
        osems = rest[2 * NB : 3 * NB]
        isem = rest[3 * NB]
        wid = lax.axis_index("s") * NUM_CORES + lax.axis_index("c")
        s0 = wid * s_per_w

        def chunk_bs(i):
            b, sub = divmod(i, n_sub)
            return b, s0 + sub * CH

        def gather(i, kb):
            return pltpu.async_copy(tok_hbm.at[idx_v.at[i]], bufs[kb], gsems[kb])

        def write_out(i, kb):
            b, col = chunk_bs(i)
            return pltpu.async_copy(
                bufs[kb], out_hbm.at[b, pl.ds(col, CH)], osems[kb]
            )

        idx_copies = [
            pltpu.async_copy(
                seq_hbm.at[chunk_bs(i)[0], pl.ds(chunk_bs(i)[1], CH)],
                idx_v.at[i],
                isem,
            )
            for i in range(n_chunks)
        ]
        pos_copy = pltpu.async_copy(pos_hbm.at[pl.ds(s0, s_per_w)], pos_v, isem)
        for c in idx_copies:
            c.wait()
        pend_w = [None] * NB
        pend_g = [None] * NB
        pend_g[0] = gather(0, 0)
        pos_copy.wait()
        for i in range(n_chunks):
            kb = i % NB
            if i + 1 < n_chunks:
                k2 = (i + 1) % NB
                if pend_w[k2] is not None:
                    pend_w[k2].wait()
                pend_g[k2] = gather(i + 1, k2)
            pend_g[kb].wait()

            p0 = (i % n_sub) * CH

            @plsc.parallel_loop(0, CH, step=1, unroll=2)
            def add_row(r, kb=kb, p0=p0):
                for c in range(D // LANES):
                    sl = pl.ds(c * LANES, LANES)
                    plsc.addupdate(bufs[kb].at[r, sl], pos_v[p0 + r, sl])

            pend_w[kb] = write_out(i, kb)
        for kb in range(NB):
            if pend_w[kb] is not None:
                pend_w[kb].wait()

    return k


def kernel(seq, tok_table, pos_table):
    B, S = seq.shape
    V, D = tok_table.shape
    k = _make_kernel(B, S, D)
    return k(seq.astype(jnp.int32), tok_table, pos_table)
